# baseline (speedup 1.0000x reference)
import functools

import jax
import jax.numpy as jnp
from jax import lax
from jax.experimental import pallas as pl
from jax.experimental.pallas import tpu as pltpu
from jax.experimental.pallas import tpu_sc as plsc

HIDDEN = 768
NUM_PROPS = 32
NUM_EXPERTS = 8
BATCH = 2048
OUT_DIM = NUM_PROPS * 2

_NW = 32
_TPW = BATCH // _NW
_CHUNK = 16


def _dot(a, b):
    return jax.lax.dot_general(
        a.astype(jnp.bfloat16), b.astype(jnp.bfloat16), (((1,), (0,)), ((), ())),
        preferred_element_type=jnp.float32)


def _router_body(mm_ref, qf_ref, rW1_ref, rb1_ref, rW2_ref, rb2_ref, logits_ref):
    h = _dot(mm_ref[...], rW1_ref[:HIDDEN]) + _dot(qf_ref[...], rW1_ref[HIDDEN:])
    h = jnp.maximum(h + rb1_ref[...], 0.0)
    logits_ref[...] = _dot(h, rW2_ref[...]) + rb2_ref[...]


_GDN = lax.GatherDimensionNumbers(
    offset_dims=(), collapsed_slice_dims=(0,), start_index_map=(0,))


def _shuf(v, idx):
    return lax.gather(v, idx[:, None], _GDN, slice_sizes=(1,),
                      mode=lax.GatherScatterMode.PROMISE_IN_BOUNDS)


def _topk_sc_body(logits_hbm, comb_hbm, lvm, ovm):
    wid = lax.axis_index("s") * 2 + lax.axis_index("c")
    base = wid * _TPW * NUM_EXPERTS
    pltpu.sync_copy(logits_hbm.at[pl.ds(base, _TPW * NUM_EXPERTS)], lvm)
    lanes = lax.iota(jnp.int32, _CHUNK)
    xor_idx = [jnp.bitwise_xor(lanes, d) for d in (1, 2, 4)]
    li = jnp.bitwise_and(lanes, NUM_EXPERTS - 1)

    def gmax(v):
        for idx in xor_idx:
            v = jnp.maximum(v, _shuf(v, idx))
        return v

    def gmin(v):
        for idx in xor_idx:
            v = jnp.minimum(v, _shuf(v, idx))
        return v

    def gsum(v):
        for idx in xor_idx:
            v = v + _shuf(v, idx)
        return v

    for chunk in range(_TPW * NUM_EXPERTS // _CHUNK):
        l = lvm[pl.ds(chunk * _CHUNK, _CHUNK)]
        m = gmax(l)
        ex = jnp.exp(l - m)
        p = ex / gsum(ex)
        w1 = gmax(p)
        c1 = gmin(jnp.where(p >= w1, li, NUM_EXPERTS))
        oh1 = li == c1
        pm = jnp.where(oh1, -1.0, p)
        w2 = gmax(pm)
        c2 = gmin(jnp.where(pm >= w2, li, NUM_EXPERTS))
        oh2 = li == c2
        denom = w1 + w2 + 1e-6
        comb = (jnp.where(oh1, w1, 0.0) + jnp.where(oh2, w2, 0.0)) / denom
        ovm[pl.ds(chunk * _CHUNK, _CHUNK)] = comb
    pltpu.sync_copy(ovm, comb_hbm.at[pl.ds(base, _TPW * NUM_EXPERTS)])


def _expert_body(mm_ref, eW1_ref, eb1_ref, eW2_ref, eb2_ref, eo_ref, xbf_ref):
    e = pl.program_id(0)

    @pl.when(e == 0)
    def _prep():
        xbf_ref[...] = mm_ref[...].astype(jnp.bfloat16)

    he = jnp.maximum(
        _dot(xbf_ref[...], eW1_ref[0]).astype(jnp.bfloat16)
        + eb1_ref[0].astype(jnp.bfloat16), 0.0)
    eo_ref[0] = _dot(he, eW2_ref[0]) + eb2_ref[0]


def _combine_body(comb_ref, eo_ref, out_ref):
    col = jax.lax.broadcasted_iota(jnp.int32, (BATCH, NUM_EXPERTS), 1)
    acc = jnp.zeros((BATCH, OUT_DIM), jnp.float32)
    for e in range(NUM_EXPERTS):
        w_col = jnp.sum(jnp.where(col == e, comb_ref[...], 0.0),
                        axis=-1, keepdims=True)
        acc = acc + w_col * eo_ref[e]
    out_ref[...] = jax.nn.sigmoid(acc)


@jax.jit
def kernel(multimodal_feat, query_feat, rW1, rb1, rW2, rb2, eW1, eb1, eW2, eb2):
    logits = pl.pallas_call(
        _router_body,
        in_specs=[
            pl.BlockSpec((BATCH, HIDDEN), lambda: (0, 0)),
            pl.BlockSpec((BATCH, HIDDEN), lambda: (0, 0)),
            pl.BlockSpec((2 * HIDDEN, HIDDEN), lambda: (0, 0)),
            pl.BlockSpec((1, HIDDEN), lambda: (0, 0)),
            pl.BlockSpec((HIDDEN, NUM_EXPERTS), lambda: (0, 0)),
            pl.BlockSpec((1, NUM_EXPERTS), lambda: (0, 0)),
        ],
        out_specs=pl.BlockSpec((BATCH, NUM_EXPERTS), lambda: (0, 0)),
        out_shape=jax.ShapeDtypeStruct((BATCH, NUM_EXPERTS), jnp.float32),
    )(multimodal_feat, query_feat, rW1, rb1.reshape(1, HIDDEN), rW2,
      rb2.reshape(1, NUM_EXPERTS))

    topk_sc = functools.partial(
        pl.kernel,
        out_type=jax.ShapeDtypeStruct((BATCH * NUM_EXPERTS,), jnp.float32),
        mesh=plsc.VectorSubcoreMesh(core_axis_name="c", subcore_axis_name="s"),
        scratch_types=[
            pltpu.VMEM((_TPW * NUM_EXPERTS,), jnp.float32),
            pltpu.VMEM((_TPW * NUM_EXPERTS,), jnp.float32),
        ],
    )(_topk_sc_body)
    comb = topk_sc(logits.reshape(BATCH * NUM_EXPERTS)).reshape(BATCH, NUM_EXPERTS)

    eo = pl.pallas_call(
        _expert_body,
        grid=(NUM_EXPERTS,),
        in_specs=[
            pl.BlockSpec((BATCH, HIDDEN), lambda e: (0, 0)),
            pl.BlockSpec((1, HIDDEN, HIDDEN), lambda e: (e, 0, 0)),
            pl.BlockSpec((1, 1, HIDDEN), lambda e: (e, 0, 0)),
            pl.BlockSpec((1, HIDDEN, OUT_DIM), lambda e: (e, 0, 0)),
            pl.BlockSpec((1, 1, OUT_DIM), lambda e: (e, 0, 0)),
        ],
        out_specs=pl.BlockSpec((1, BATCH, OUT_DIM), lambda e: (e, 0, 0)),
        out_shape=jax.ShapeDtypeStruct((NUM_EXPERTS, BATCH, OUT_DIM), jnp.float32),
        scratch_shapes=[
            pltpu.VMEM((BATCH, HIDDEN), jnp.bfloat16),
        ],
        compiler_params=pltpu.CompilerParams(
            dimension_semantics=("arbitrary",),
        ),
    )(multimodal_feat, eW1,
      eb1.reshape(NUM_EXPERTS, 1, HIDDEN), eW2,
      eb2.reshape(NUM_EXPERTS, 1, OUT_DIM))

    out = pl.pallas_call(
        _combine_body,
        in_specs=[
            pl.BlockSpec((BATCH, NUM_EXPERTS), lambda: (0, 0)),
            pl.BlockSpec((NUM_EXPERTS, BATCH, OUT_DIM), lambda: (0, 0, 0)),
        ],
        out_specs=pl.BlockSpec((BATCH, OUT_DIM), lambda: (0, 0)),
        out_shape=jax.ShapeDtypeStruct((BATCH, OUT_DIM), jnp.float32),
    )(comb, eo)
    n = BATCH * NUM_PROPS
    ev = out[:, 0::2].reshape(n, 1)
    od = out[:, 1::2].reshape(n, 1)
    return jnp.concatenate([ev, od], axis=1)


# speedup vs baseline: 1.2757x; 1.2757x over previous
import jax
import jax.numpy as jnp
from jax.experimental import pallas as pl
from jax.experimental.pallas import tpu as pltpu

HIDDEN = 768
NUM_PROPS = 32
NUM_EXPERTS = 8
BATCH = 2048
OUT_DIM = NUM_PROPS * 2


def _dot(a, b):
    return jax.lax.dot_general(
        a.astype(jnp.bfloat16), b.astype(jnp.bfloat16), (((1,), (0,)), ((), ())),
        preferred_element_type=jnp.float32)


def _router_body(mm_ref, qf_ref, rW1_ref, rb1_ref, rW2_ref, rb2_ref, comb_ref):
    col = jax.lax.broadcasted_iota(jnp.int32, (BATCH, NUM_EXPERTS), 1)
    h = _dot(mm_ref[...], rW1_ref[:HIDDEN]) + _dot(qf_ref[...], rW1_ref[HIDDEN:])
    h = jnp.maximum(h + rb1_ref[...], 0.0)
    logits = _dot(h, rW2_ref[...]) + rb2_ref[...]
    m = jnp.max(logits, axis=-1, keepdims=True)
    ex = jnp.exp(logits - m)
    p = ex / jnp.sum(ex, axis=-1, keepdims=True)
    w1 = jnp.max(p, axis=-1, keepdims=True)
    c1 = jnp.min(jnp.where(p >= w1, col, NUM_EXPERTS), axis=-1, keepdims=True)
    oh1 = col == c1
    pm = jnp.where(oh1, -jnp.inf, p)
    w2 = jnp.max(pm, axis=-1, keepdims=True)
    c2 = jnp.min(jnp.where(pm >= w2, col, NUM_EXPERTS), axis=-1, keepdims=True)
    oh2 = col == c2
    denom = w1 + w2 + 1e-6
    comb_ref[...] = (jnp.where(oh1, w1, 0.0) + jnp.where(oh2, w2, 0.0)) / denom


def _expert_body(mm_ref, comb_ref, eW1_ref, eb1_ref, eW2_ref, eb2_ref,
                 out_ref, xbf_ref, acc_ref):
    e = pl.program_id(0)
    col = jax.lax.broadcasted_iota(jnp.int32, (BATCH, NUM_EXPERTS), 1)

    @pl.when(e == 0)
    def _prep():
        xbf_ref[...] = mm_ref[...].astype(jnp.bfloat16)

    he = jnp.maximum(
        _dot(xbf_ref[...], eW1_ref[0]).astype(jnp.bfloat16)
        + eb1_ref[0].astype(jnp.bfloat16), 0.0)
    o = _dot(he, eW2_ref[0]) + eb2_ref[0]
    w_col = jnp.sum(jnp.where(col == e, comb_ref[...], 0.0), axis=-1, keepdims=True)
    contrib = w_col * o

    @pl.when(e == 0)
    def _init():
        acc_ref[...] = contrib

    @pl.when(e > 0)
    def _acc():
        acc_ref[...] += contrib

    @pl.when(e == NUM_EXPERTS - 1)
    def _fin():
        out_ref[...] = jax.nn.sigmoid(acc_ref[...])


@jax.jit
def kernel(multimodal_feat, query_feat, rW1, rb1, rW2, rb2, eW1, eb1, eW2, eb2):
    c2 = lambda: (0, 0)
    comb = pl.pallas_call(
        _router_body,
        in_specs=[
            pl.BlockSpec((BATCH, HIDDEN), lambda: (0, 0)),
            pl.BlockSpec((BATCH, HIDDEN), lambda: (0, 0)),
            pl.BlockSpec((2 * HIDDEN, HIDDEN), lambda: (0, 0)),
            pl.BlockSpec((1, HIDDEN), lambda: (0, 0)),
            pl.BlockSpec((HIDDEN, NUM_EXPERTS), lambda: (0, 0)),
            pl.BlockSpec((1, NUM_EXPERTS), lambda: (0, 0)),
        ],
        out_specs=pl.BlockSpec((BATCH, NUM_EXPERTS), lambda: (0, 0)),
        out_shape=jax.ShapeDtypeStruct((BATCH, NUM_EXPERTS), jnp.float32),
    )(multimodal_feat, query_feat, rW1, rb1.reshape(1, HIDDEN), rW2,
      rb2.reshape(1, NUM_EXPERTS))

    out = pl.pallas_call(
        _expert_body,
        grid=(NUM_EXPERTS,),
        in_specs=[
            pl.BlockSpec((BATCH, HIDDEN), lambda e: (0, 0)),
            pl.BlockSpec((BATCH, NUM_EXPERTS), lambda e: (0, 0)),
            pl.BlockSpec((1, HIDDEN, HIDDEN), lambda e: (e, 0, 0)),
            pl.BlockSpec((1, 1, HIDDEN), lambda e: (e, 0, 0)),
            pl.BlockSpec((1, HIDDEN, OUT_DIM), lambda e: (e, 0, 0)),
            pl.BlockSpec((1, 1, OUT_DIM), lambda e: (e, 0, 0)),
        ],
        out_specs=pl.BlockSpec((BATCH, OUT_DIM), lambda e: (0, 0)),
        out_shape=jax.ShapeDtypeStruct((BATCH, OUT_DIM), jnp.float32),
        scratch_shapes=[
            pltpu.VMEM((BATCH, HIDDEN), jnp.bfloat16),
            pltpu.VMEM((BATCH, OUT_DIM), jnp.float32),
        ],
        compiler_params=pltpu.CompilerParams(
            dimension_semantics=("arbitrary",),
        ),
    )(multimodal_feat, comb, eW1,
      eb1.reshape(NUM_EXPERTS, 1, HIDDEN), eW2,
      eb2.reshape(NUM_EXPERTS, 1, OUT_DIM))
    n = BATCH * NUM_PROPS
    ev = out[:, 0::2].reshape(n, 1)
    od = out[:, 1::2].reshape(n, 1)
    return jnp.concatenate([ev, od], axis=1)
